# baseline (speedup 1.0000x reference)
import jax
import jax.numpy as jnp
from jax import lax
from jax.experimental import pallas as pl
from jax.experimental.pallas import tpu as pltpu
from jax.experimental.pallas import tpu_sc as plsc

BATCH = 4096
MAX_LEN = 200
EMB_DIM = 64
OUT_DIM = 8
VROWS = 1000002

L = 16
G0 = 128
G1 = 72
PADW = 256
NW = 32
BPW = BATCH // NW

KB = 512
NB = 977
H = NB * KB


def _pack_body(a_ref, b_ref, o_ref):
    o_ref[:, 0:EMB_DIM] = jnp.transpose(a_ref[:])
    o_ref[:, EMB_DIM:128] = jnp.transpose(b_ref[:])


def _tc_pack(tT):
    return pl.pallas_call(
        _pack_body,
        grid=(NB,),
        in_specs=[
            pl.BlockSpec((EMB_DIM, KB), lambda i: (0, i)),
            pl.BlockSpec((EMB_DIM, KB), lambda i: (0, NB + i)),
        ],
        out_specs=pl.BlockSpec((KB, 128), lambda i: (i, 0)),
        out_shape=jax.ShapeDtypeStruct((H, 128), jnp.float32),
    )(tT, tT)


def _relayout_body(d_ref, o_ref):
    x = d_ref[:]
    x = jnp.where(x < H, 2 * x, 2 * x - (2 * H - 1))
    x = jnp.concatenate(
        [x, jnp.zeros((BATCH, PADW - MAX_LEN), jnp.int32)], axis=1)
    o_ref[:] = jnp.reshape(x, (BATCH * 2, 128))


def _tc_relayout(data_i):
    return pl.pallas_call(
        _relayout_body,
        out_shape=jax.ShapeDtypeStruct((BATCH * 2, 128), jnp.int32),
    )(data_i)


def _sc_body(dlin, table, out_hbm, idxblk,
             rows0a, rows1a, rows0b, rows1b, outbuf, sema, semb):
    nc = 2
    wid = lax.axis_index("s") * nc + lax.axis_index("c")
    base = wid * BPW

    pltpu.sync_copy(dlin.at[pl.ds(2 * base, 2 * BPW), :], idxblk)

    def issue(b, r0, r1, sem):
        pltpu.async_copy(table.at[idxblk.at[2 * b]], r0, sem)
        pltpu.async_copy(table.at[idxblk.at[2 * b + 1, pl.ds(0, G1)]], r1, sem)

    def wait(r0, r1, sem):
        pltpu.make_async_copy(table.at[idxblk.at[0]], r0, sem).wait()
        pltpu.make_async_copy(table.at[idxblk.at[1, pl.ds(0, G1)]], r1, sem).wait()

    def consume(b, r0, r1):
        def acc2(j, acc):
            a0, a1, a2, a3 = acc
            a0 = a0 + r0[j, pl.ds(0 * L, L)] + r1[j, pl.ds(0 * L, L)]
            a1 = a1 + r0[j, pl.ds(1 * L, L)] + r1[j, pl.ds(1 * L, L)]
            a2 = a2 + r0[j, pl.ds(2 * L, L)] + r1[j, pl.ds(2 * L, L)]
            a3 = a3 + r0[j, pl.ds(3 * L, L)] + r1[j, pl.ds(3 * L, L)]
            return (a0, a1, a2, a3)

        def acc1(j, acc):
            a0, a1, a2, a3 = acc
            a0 = a0 + r0[j, pl.ds(0 * L, L)]
            a1 = a1 + r0[j, pl.ds(1 * L, L)]
            a2 = a2 + r0[j, pl.ds(2 * L, L)]
            a3 = a3 + r0[j, pl.ds(3 * L, L)]
            return (a0, a1, a2, a3)

        zf = jnp.zeros((L,), jnp.float32)
        acc = lax.fori_loop(0, G1, acc2, (zf, zf, zf, zf))
        acc = lax.fori_loop(G1, G0, acc1, acc)
        for d in range(4):
            outbuf[pl.ds(b * EMB_DIM + d * L, L)] = acc[d]

    issue(0, rows0a, rows1a, sema)

    def per_pair(g, _):
        b0 = pl.multiple_of(g * 2, 2)
        issue(b0 + 1, rows0b, rows1b, semb)
        wait(rows0a, rows1a, sema)
        consume(b0, rows0a, rows1a)

        @pl.when(g < BPW // 2 - 1)
        def _issue_next():
            issue(b0 + 2, rows0a, rows1a, sema)

        wait(rows0b, rows1b, semb)
        consume(b0 + 1, rows0b, rows1b)
        return _

    lax.fori_loop(0, BPW // 2, per_pair, None)
    pltpu.sync_copy(outbuf, out_hbm.at[pl.ds(base * EMB_DIM, BPW * EMB_DIM)])


def _sc_gather_sum(dlin, table):
    mesh = plsc.VectorSubcoreMesh(core_axis_name="c", subcore_axis_name="s")
    return pl.kernel(
        _sc_body,
        mesh=mesh,
        compiler_params=pltpu.CompilerParams(use_tc_tiling_on_sc=False),
        out_type=jax.ShapeDtypeStruct((BATCH * EMB_DIM,), jnp.float32),
        scratch_types=[
            pltpu.VMEM((2 * BPW, 128), jnp.int32),
            pltpu.VMEM((G0, EMB_DIM), jnp.float32),
            pltpu.VMEM((G1, EMB_DIM), jnp.float32),
            pltpu.VMEM((G0, EMB_DIM), jnp.float32),
            pltpu.VMEM((G1, EMB_DIM), jnp.float32),
            pltpu.VMEM((BPW * EMB_DIM,), jnp.float32),
            pltpu.SemaphoreType.DMA,
            pltpu.SemaphoreType.DMA,
        ],
    )(dlin, table)


def _finish_body(s_ref, d_ref, r0_ref, l_ref, w_ref, b_ref, o_ref):
    cnt = jnp.sum(jnp.where(d_ref[:] == 0, 1.0, 0.0), axis=1, keepdims=True)
    x = (s_ref[:] - cnt * r0_ref[:]) / l_ref[:]
    o_ref[:] = jnp.dot(x, w_ref[:], preferred_element_type=jnp.float32) + b_ref[:]


def _tc_finish(sums, data_i, row0, lenf, w1t, b1r):
    return pl.pallas_call(
        _finish_body,
        out_shape=jax.ShapeDtypeStruct((BATCH, OUT_DIM), jnp.float32),
    )(sums, data_i, row0, lenf, w1t, b1r)


def kernel(data, length, embed_table, W1, b1):
    data_i = data.astype(jnp.int32)
    pack = _tc_pack(embed_table.T)
    dlin = _tc_relayout(data_i)
    sums = _sc_gather_sum(dlin, pack.reshape(2 * H, EMB_DIM)).reshape(
        BATCH, EMB_DIM)
    lenf = length.astype(jnp.float32).reshape(BATCH, 1)
    return _tc_finish(sums, data_i, embed_table[0:1], lenf, W1.T,
                      b1.reshape(1, OUT_DIM))


# speedup vs baseline: 1.7727x; 1.7727x over previous
import jax
import jax.numpy as jnp
from jax import lax
from jax.experimental import pallas as pl
from jax.experimental.pallas import tpu as pltpu
from jax.experimental.pallas import tpu_sc as plsc

BATCH = 4096
MAX_LEN = 200
EMB_DIM = 64
OUT_DIM = 8
VROWS = 1000002

L = 16
G0 = 128
G1 = 72
PADW = 256
NW = 32
BPW = BATCH // NW

KB = 2048
NB = 245
H = NB * KB


def _pack_body(a_ref, b_ref, o_ref):
    o_ref[:, 0:EMB_DIM] = jnp.transpose(a_ref[:])
    o_ref[:, EMB_DIM:128] = jnp.transpose(b_ref[:])


def _tc_pack(tT):
    return pl.pallas_call(
        _pack_body,
        grid=(NB,),
        in_specs=[
            pl.BlockSpec((EMB_DIM, KB), lambda i: (0, i)),
            pl.BlockSpec(
                (EMB_DIM, KB),
                lambda i: (0, jnp.minimum(NB + i, (VROWS - 1) // KB)),
            ),
        ],
        out_specs=pl.BlockSpec((KB, 128), lambda i: (i, 0)),
        out_shape=jax.ShapeDtypeStruct((H, 128), jnp.float32),
    )(tT, tT)


def _relayout_body(d_ref, o_ref):
    x = d_ref[:]
    x = jnp.where(x < H, 2 * x, 2 * x - (2 * H - 1))
    x = jnp.concatenate(
        [x, jnp.zeros((BATCH, PADW - MAX_LEN), jnp.int32)], axis=1)
    o_ref[:] = jnp.reshape(x, (BATCH * 2, 128))


def _tc_relayout(data_i):
    return pl.pallas_call(
        _relayout_body,
        out_shape=jax.ShapeDtypeStruct((BATCH * 2, 128), jnp.int32),
    )(data_i)


def _sc_body(dlin, table, out_hbm, idxblk,
             rows0a, rows1a, rows0b, rows1b, outbuf, sema, semb):
    nc = 2
    wid = lax.axis_index("s") * nc + lax.axis_index("c")
    base = wid * BPW

    pltpu.sync_copy(dlin.at[pl.ds(2 * base, 2 * BPW), :], idxblk)

    def issue(b, r0, r1, sem):
        pltpu.async_copy(table.at[idxblk.at[2 * b]], r0, sem)
        pltpu.async_copy(table.at[idxblk.at[2 * b + 1, pl.ds(0, G1)]], r1, sem)

    def wait(r0, r1, sem):
        pltpu.make_async_copy(table.at[idxblk.at[0]], r0, sem).wait()
        pltpu.make_async_copy(table.at[idxblk.at[1, pl.ds(0, G1)]], r1, sem).wait()

    def consume(b, r0, r1):
        def acc2(j, acc):
            a0, a1, a2, a3 = acc
            a0 = a0 + r0[j, pl.ds(0 * L, L)] + r1[j, pl.ds(0 * L, L)]
            a1 = a1 + r0[j, pl.ds(1 * L, L)] + r1[j, pl.ds(1 * L, L)]
            a2 = a2 + r0[j, pl.ds(2 * L, L)] + r1[j, pl.ds(2 * L, L)]
            a3 = a3 + r0[j, pl.ds(3 * L, L)] + r1[j, pl.ds(3 * L, L)]
            return (a0, a1, a2, a3)

        def acc1(j, acc):
            a0, a1, a2, a3 = acc
            a0 = a0 + r0[j, pl.ds(0 * L, L)]
            a1 = a1 + r0[j, pl.ds(1 * L, L)]
            a2 = a2 + r0[j, pl.ds(2 * L, L)]
            a3 = a3 + r0[j, pl.ds(3 * L, L)]
            return (a0, a1, a2, a3)

        zf = jnp.zeros((L,), jnp.float32)
        acc = lax.fori_loop(0, G1, acc2, (zf, zf, zf, zf))
        acc = lax.fori_loop(G1, G0, acc1, acc)
        for d in range(4):
            outbuf[pl.ds(b * EMB_DIM + d * L, L)] = acc[d]

    issue(0, rows0a, rows1a, sema)

    def per_pair(g, _):
        b0 = pl.multiple_of(g * 2, 2)
        issue(b0 + 1, rows0b, rows1b, semb)
        wait(rows0a, rows1a, sema)
        consume(b0, rows0a, rows1a)

        @pl.when(g < BPW // 2 - 1)
        def _issue_next():
            issue(b0 + 2, rows0a, rows1a, sema)

        wait(rows0b, rows1b, semb)
        consume(b0 + 1, rows0b, rows1b)
        return _

    lax.fori_loop(0, BPW // 2, per_pair, None)
    pltpu.sync_copy(outbuf, out_hbm.at[pl.ds(base * EMB_DIM, BPW * EMB_DIM)])


def _sc_gather_sum(dlin, table):
    mesh = plsc.VectorSubcoreMesh(core_axis_name="c", subcore_axis_name="s")
    return pl.kernel(
        _sc_body,
        mesh=mesh,
        compiler_params=pltpu.CompilerParams(use_tc_tiling_on_sc=False),
        out_type=jax.ShapeDtypeStruct((BATCH * EMB_DIM,), jnp.float32),
        scratch_types=[
            pltpu.VMEM((2 * BPW, 128), jnp.int32),
            pltpu.VMEM((G0, EMB_DIM), jnp.float32),
            pltpu.VMEM((G1, EMB_DIM), jnp.float32),
            pltpu.VMEM((G0, EMB_DIM), jnp.float32),
            pltpu.VMEM((G1, EMB_DIM), jnp.float32),
            pltpu.VMEM((BPW * EMB_DIM,), jnp.float32),
            pltpu.SemaphoreType.DMA,
            pltpu.SemaphoreType.DMA,
        ],
    )(dlin, table)


def _finish_body(s_ref, d_ref, r0_ref, l_ref, w_ref, b_ref, o_ref):
    cnt = jnp.sum(jnp.where(d_ref[:] == 0, 1.0, 0.0), axis=1, keepdims=True)
    x = (s_ref[:] - cnt * r0_ref[:]) / l_ref[:]
    o_ref[:] = jnp.dot(x, w_ref[:], preferred_element_type=jnp.float32) + b_ref[:]


def _tc_finish(sums, data_i, row0, lenf, w1t, b1r):
    return pl.pallas_call(
        _finish_body,
        out_shape=jax.ShapeDtypeStruct((BATCH, OUT_DIM), jnp.float32),
    )(sums, data_i, row0, lenf, w1t, b1r)


def kernel(data, length, embed_table, W1, b1):
    data_i = data.astype(jnp.int32)
    pack = _tc_pack(embed_table.T)
    dlin = _tc_relayout(data_i)
    sums = _sc_gather_sum(dlin, pack.reshape(2 * H, EMB_DIM)).reshape(
        BATCH, EMB_DIM)
    lenf = length.astype(jnp.float32).reshape(BATCH, 1)
    return _tc_finish(sums, data_i, embed_table[0:1], lenf, W1.T,
                      b1.reshape(1, OUT_DIM))


# speedup vs baseline: 2.0690x; 1.1671x over previous
import jax
import jax.numpy as jnp
from jax import lax
from jax.experimental import pallas as pl
from jax.experimental.pallas import tpu as pltpu
from jax.experimental.pallas import tpu_sc as plsc

BATCH = 4096
MAX_LEN = 200
EMB_DIM = 64
OUT_DIM = 8
VROWS = 1000002

L = 16
G0 = 128
G1 = 72
PADW = 256
NW = 32
BPW = BATCH // NW

KB = 4096
NB = 123
H = NB * KB


def _pack_body(a_ref, b_ref, o_ref):
    o_ref[:, 0:EMB_DIM] = jnp.transpose(a_ref[:])
    o_ref[:, EMB_DIM:128] = jnp.transpose(b_ref[:])


def _tc_pack(tT):
    return pl.pallas_call(
        _pack_body,
        grid=(NB,),
        in_specs=[
            pl.BlockSpec((EMB_DIM, KB), lambda i: (0, i)),
            pl.BlockSpec(
                (EMB_DIM, KB),
                lambda i: (0, jnp.minimum(NB + i, (VROWS - 1) // KB)),
            ),
        ],
        out_specs=pl.BlockSpec((KB, 128), lambda i: (i, 0)),
        out_shape=jax.ShapeDtypeStruct((H, 128), jnp.float32),
    )(tT, tT)


def _relayout_body(d_ref, o_ref):
    x = d_ref[:]
    x = jnp.where(x < H, 2 * x, 2 * x - (2 * H - 1))
    x = jnp.concatenate(
        [x, jnp.zeros((BATCH, PADW - MAX_LEN), jnp.int32)], axis=1)
    o_ref[:] = jnp.reshape(x, (BATCH * 2, 128))


def _tc_relayout(data_i):
    return pl.pallas_call(
        _relayout_body,
        out_shape=jax.ShapeDtypeStruct((BATCH * 2, 128), jnp.int32),
    )(data_i)


def _sc_body(dlin, table, out_hbm, idxblk,
             rows0a, rows1a, rows0b, rows1b, outbuf, sema, semb):
    nc = 2
    wid = lax.axis_index("s") * nc + lax.axis_index("c")
    base = wid * BPW

    pltpu.sync_copy(dlin.at[pl.ds(2 * base, 2 * BPW), :], idxblk)

    def issue(b, r0, r1, sem):
        pltpu.async_copy(table.at[idxblk.at[2 * b]], r0, sem)
        pltpu.async_copy(table.at[idxblk.at[2 * b + 1, pl.ds(0, G1)]], r1, sem)

    def wait(r0, r1, sem):
        pltpu.make_async_copy(table.at[idxblk.at[0]], r0, sem).wait()
        pltpu.make_async_copy(table.at[idxblk.at[1, pl.ds(0, G1)]], r1, sem).wait()

    def consume(b, r0, r1):
        def acc2(j, acc):
            a0, a1, a2, a3 = acc
            a0 = a0 + r0[j, pl.ds(0 * L, L)] + r1[j, pl.ds(0 * L, L)]
            a1 = a1 + r0[j, pl.ds(1 * L, L)] + r1[j, pl.ds(1 * L, L)]
            a2 = a2 + r0[j, pl.ds(2 * L, L)] + r1[j, pl.ds(2 * L, L)]
            a3 = a3 + r0[j, pl.ds(3 * L, L)] + r1[j, pl.ds(3 * L, L)]
            return (a0, a1, a2, a3)

        def acc1(j, acc):
            a0, a1, a2, a3 = acc
            a0 = a0 + r0[j, pl.ds(0 * L, L)]
            a1 = a1 + r0[j, pl.ds(1 * L, L)]
            a2 = a2 + r0[j, pl.ds(2 * L, L)]
            a3 = a3 + r0[j, pl.ds(3 * L, L)]
            return (a0, a1, a2, a3)

        zf = jnp.zeros((L,), jnp.float32)
        acc = lax.fori_loop(0, G1, acc2, (zf, zf, zf, zf))
        acc = lax.fori_loop(G1, G0, acc1, acc)
        for d in range(4):
            outbuf[pl.ds(b * EMB_DIM + d * L, L)] = acc[d]

    issue(0, rows0a, rows1a, sema)

    def per_pair(g, _):
        b0 = pl.multiple_of(g * 2, 2)
        issue(b0 + 1, rows0b, rows1b, semb)
        wait(rows0a, rows1a, sema)
        consume(b0, rows0a, rows1a)

        @pl.when(g < BPW // 2 - 1)
        def _issue_next():
            issue(b0 + 2, rows0a, rows1a, sema)

        wait(rows0b, rows1b, semb)
        consume(b0 + 1, rows0b, rows1b)
        return _

    lax.fori_loop(0, BPW // 2, per_pair, None)
    pltpu.sync_copy(outbuf, out_hbm.at[pl.ds(base * EMB_DIM, BPW * EMB_DIM)])


def _sc_gather_sum(dlin, table):
    mesh = plsc.VectorSubcoreMesh(core_axis_name="c", subcore_axis_name="s")
    return pl.kernel(
        _sc_body,
        mesh=mesh,
        compiler_params=pltpu.CompilerParams(use_tc_tiling_on_sc=False),
        out_type=jax.ShapeDtypeStruct((BATCH * EMB_DIM,), jnp.float32),
        scratch_types=[
            pltpu.VMEM((2 * BPW, 128), jnp.int32),
            pltpu.VMEM((G0, EMB_DIM), jnp.float32),
            pltpu.VMEM((G1, EMB_DIM), jnp.float32),
            pltpu.VMEM((G0, EMB_DIM), jnp.float32),
            pltpu.VMEM((G1, EMB_DIM), jnp.float32),
            pltpu.VMEM((BPW * EMB_DIM,), jnp.float32),
            pltpu.SemaphoreType.DMA,
            pltpu.SemaphoreType.DMA,
        ],
    )(dlin, table)


def _finish_body(s_ref, d_ref, r0_ref, l_ref, w_ref, b_ref, o_ref):
    cnt = jnp.sum(jnp.where(d_ref[:] == 0, 1.0, 0.0), axis=1, keepdims=True)
    x = (s_ref[:] - cnt * r0_ref[:]) / l_ref[:]
    o_ref[:] = jnp.dot(x, w_ref[:], preferred_element_type=jnp.float32) + b_ref[:]


def _tc_finish(sums, data_i, row0, lenf, w1t, b1r):
    return pl.pallas_call(
        _finish_body,
        out_shape=jax.ShapeDtypeStruct((BATCH, OUT_DIM), jnp.float32),
    )(sums, data_i, row0, lenf, w1t, b1r)


def kernel(data, length, embed_table, W1, b1):
    data_i = data.astype(jnp.int32)
    pack = _tc_pack(embed_table.T)
    dlin = _tc_relayout(data_i)
    sums = _sc_gather_sum(dlin, pack.reshape(2 * H, EMB_DIM)).reshape(
        BATCH, EMB_DIM)
    lenf = length.astype(jnp.float32).reshape(BATCH, 1)
    return _tc_finish(sums, data_i, embed_table[0:1], lenf, W1.T,
                      b1.reshape(1, OUT_DIM))


# speedup vs baseline: 2.2439x; 1.0845x over previous
import jax
import jax.numpy as jnp
from jax import lax
from jax.experimental import pallas as pl
from jax.experimental.pallas import tpu as pltpu
from jax.experimental.pallas import tpu_sc as plsc

BATCH = 4096
MAX_LEN = 200
EMB_DIM = 64
OUT_DIM = 8
VROWS = 1000002

L = 16
G0 = 128
G1 = 72
PADW = 256
NW = 32
BPW = BATCH // NW

KB = 8192
NB = 62
H = NB * KB


def _pack_body(a_ref, b_ref, o_ref):
    o_ref[:, 0:EMB_DIM] = jnp.transpose(a_ref[:])
    o_ref[:, EMB_DIM:128] = jnp.transpose(b_ref[:])


def _tc_pack(tT):
    return pl.pallas_call(
        _pack_body,
        grid=(NB,),
        in_specs=[
            pl.BlockSpec((EMB_DIM, KB), lambda i: (0, i)),
            pl.BlockSpec(
                (EMB_DIM, KB),
                lambda i: (0, jnp.minimum(NB + i, (VROWS - 1) // KB)),
            ),
        ],
        out_specs=pl.BlockSpec((KB, 128), lambda i: (i, 0)),
        out_shape=jax.ShapeDtypeStruct((H, 128), jnp.float32),
    )(tT, tT)


def _relayout_body(d_ref, o_ref):
    x = d_ref[:]
    x = jnp.where(x < H, 2 * x, 2 * x - (2 * H - 1))
    x = jnp.concatenate(
        [x, jnp.zeros((BATCH, PADW - MAX_LEN), jnp.int32)], axis=1)
    o_ref[:] = jnp.reshape(x, (BATCH * 2, 128))


def _tc_relayout(data_i):
    return pl.pallas_call(
        _relayout_body,
        out_shape=jax.ShapeDtypeStruct((BATCH * 2, 128), jnp.int32),
    )(data_i)


def _sc_body(dlin, table, out_hbm, idxblk,
             rows0a, rows1a, rows0b, rows1b, outbuf, sema, semb):
    nc = 2
    wid = lax.axis_index("s") * nc + lax.axis_index("c")
    base = wid * BPW

    pltpu.sync_copy(dlin.at[pl.ds(2 * base, 2 * BPW), :], idxblk)

    def issue(b, r0, r1, sem):
        pltpu.async_copy(table.at[idxblk.at[2 * b]], r0, sem)
        pltpu.async_copy(table.at[idxblk.at[2 * b + 1, pl.ds(0, G1)]], r1, sem)

    def wait(r0, r1, sem):
        pltpu.make_async_copy(table.at[idxblk.at[0]], r0, sem).wait()
        pltpu.make_async_copy(table.at[idxblk.at[1, pl.ds(0, G1)]], r1, sem).wait()

    def consume(b, r0, r1):
        def acc2(j, acc):
            a0, a1, a2, a3 = acc
            a0 = a0 + r0[j, pl.ds(0 * L, L)] + r1[j, pl.ds(0 * L, L)]
            a1 = a1 + r0[j, pl.ds(1 * L, L)] + r1[j, pl.ds(1 * L, L)]
            a2 = a2 + r0[j, pl.ds(2 * L, L)] + r1[j, pl.ds(2 * L, L)]
            a3 = a3 + r0[j, pl.ds(3 * L, L)] + r1[j, pl.ds(3 * L, L)]
            return (a0, a1, a2, a3)

        def acc1(j, acc):
            a0, a1, a2, a3 = acc
            a0 = a0 + r0[j, pl.ds(0 * L, L)]
            a1 = a1 + r0[j, pl.ds(1 * L, L)]
            a2 = a2 + r0[j, pl.ds(2 * L, L)]
            a3 = a3 + r0[j, pl.ds(3 * L, L)]
            return (a0, a1, a2, a3)

        zf = jnp.zeros((L,), jnp.float32)
        acc = lax.fori_loop(0, G1, acc2, (zf, zf, zf, zf))
        acc = lax.fori_loop(G1, G0, acc1, acc)
        for d in range(4):
            outbuf[pl.ds(b * EMB_DIM + d * L, L)] = acc[d]

    issue(0, rows0a, rows1a, sema)

    def per_pair(g, _):
        b0 = pl.multiple_of(g * 2, 2)
        issue(b0 + 1, rows0b, rows1b, semb)
        wait(rows0a, rows1a, sema)
        consume(b0, rows0a, rows1a)

        @pl.when(g < BPW // 2 - 1)
        def _issue_next():
            issue(b0 + 2, rows0a, rows1a, sema)

        wait(rows0b, rows1b, semb)
        consume(b0 + 1, rows0b, rows1b)
        return _

    lax.fori_loop(0, BPW // 2, per_pair, None)
    pltpu.sync_copy(outbuf, out_hbm.at[pl.ds(base * EMB_DIM, BPW * EMB_DIM)])


def _sc_gather_sum(dlin, table):
    mesh = plsc.VectorSubcoreMesh(core_axis_name="c", subcore_axis_name="s")
    return pl.kernel(
        _sc_body,
        mesh=mesh,
        compiler_params=pltpu.CompilerParams(use_tc_tiling_on_sc=False),
        out_type=jax.ShapeDtypeStruct((BATCH * EMB_DIM,), jnp.float32),
        scratch_types=[
            pltpu.VMEM((2 * BPW, 128), jnp.int32),
            pltpu.VMEM((G0, EMB_DIM), jnp.float32),
            pltpu.VMEM((G1, EMB_DIM), jnp.float32),
            pltpu.VMEM((G0, EMB_DIM), jnp.float32),
            pltpu.VMEM((G1, EMB_DIM), jnp.float32),
            pltpu.VMEM((BPW * EMB_DIM,), jnp.float32),
            pltpu.SemaphoreType.DMA,
            pltpu.SemaphoreType.DMA,
        ],
    )(dlin, table)


def _finish_body(s_ref, d_ref, r0_ref, l_ref, w_ref, b_ref, o_ref):
    cnt = jnp.sum(jnp.where(d_ref[:] == 0, 1.0, 0.0), axis=1, keepdims=True)
    x = (s_ref[:] - cnt * r0_ref[:]) / l_ref[:]
    o_ref[:] = jnp.dot(x, w_ref[:], preferred_element_type=jnp.float32) + b_ref[:]


def _tc_finish(sums, data_i, row0, lenf, w1t, b1r):
    return pl.pallas_call(
        _finish_body,
        out_shape=jax.ShapeDtypeStruct((BATCH, OUT_DIM), jnp.float32),
    )(sums, data_i, row0, lenf, w1t, b1r)


def kernel(data, length, embed_table, W1, b1):
    data_i = data.astype(jnp.int32)
    pack = _tc_pack(embed_table.T)
    dlin = _tc_relayout(data_i)
    sums = _sc_gather_sum(dlin, pack.reshape(2 * H, EMB_DIM)).reshape(
        BATCH, EMB_DIM)
    lenf = length.astype(jnp.float32).reshape(BATCH, 1)
    return _tc_finish(sums, data_i, embed_table[0:1], lenf, W1.T,
                      b1.reshape(1, OUT_DIM))
